# Initial kernel scaffold; baseline (speedup 1.0000x reference)
import functools

import jax
import jax.numpy as jnp
from jax import lax
from jax.experimental import pallas as pl
from jax.experimental.pallas import tpu as pltpu
from jax.experimental.pallas import tpu_sc as plsc

N = 10000
E = 320000
D_IN = 128
D_HID = 128
D_OUT = 64

NC = 2
NS = 16
NW = NC * NS

CHUNK = 128
K = -(-E // (NW * CHUNK))
E_PAD = NW * K * CHUNK
N_PAD = K * CHUNK
RPT = N_PAD // NS

_SC_MESH = plsc.VectorSubcoreMesh(core_axis_name="c", subcore_axis_name="s")


def _counts_body(dst_hbm, zeros_hbm, out_hbm, dst_v, ones_v, cnt_sh):
    c = lax.axis_index("c")
    s = lax.axis_index("s")
    wid = c * NS + s
    pltpu.sync_copy(dst_hbm.at[wid], dst_v)
    for i in range(CHUNK // 16):
        ones_v[pl.ds(i * 16, 16)] = jnp.ones((16,), jnp.float32)
    row0 = s * RPT
    pltpu.sync_copy(zeros_hbm.at[pl.ds(row0, RPT)], cnt_sh.at[pl.ds(row0, RPT)])
    plsc.subcore_barrier()

    def body(j, carry):
        pltpu.sync_copy(ones_v, cnt_sh.at[dst_v.at[j]], add=True)
        return carry

    lax.fori_loop(0, K, body, 0)
    plsc.subcore_barrier()
    pltpu.sync_copy(cnt_sh.at[pl.ds(row0, RPT)], out_hbm.at[c, pl.ds(row0, RPT)])


@functools.partial(
    pl.kernel,
    out_type=jax.ShapeDtypeStruct((NC, N_PAD), jnp.float32),
    mesh=_SC_MESH,
    scratch_types=[
        pltpu.VMEM((K, CHUNK), jnp.int32),
        pltpu.VMEM((CHUNK,), jnp.float32),
        pltpu.VMEM_SHARED((N_PAD,), jnp.float32),
    ],
)
def _sc_counts(dst_hbm, zeros_hbm, out_hbm, dst_v, ones_v, cnt_sh):
    _counts_body(dst_hbm, zeros_hbm, out_hbm, dst_v, ones_v, cnt_sh)


def _agg_body(hp_hbm, src_hbm, dst_hbm, zeros_hbm, out_hbm,
              src_v, dst_v, rows_v, sem, acc_sh):
    c = lax.axis_index("c")
    s = lax.axis_index("s")
    wid = c * NS + s
    pltpu.sync_copy(src_hbm.at[wid], src_v)
    pltpu.sync_copy(dst_hbm.at[wid], dst_v)
    row0 = s * RPT
    pltpu.sync_copy(zeros_hbm.at[pl.ds(row0, RPT)], acc_sh.at[pl.ds(row0, RPT)])
    plsc.subcore_barrier()

    def body(j, carry):
        pltpu.async_copy(hp_hbm.at[src_v.at[j]], rows_v, sem).wait()
        pltpu.sync_copy(rows_v, acc_sh.at[dst_v.at[j]], add=True)
        return carry

    lax.fori_loop(0, K, body, 0)
    plsc.subcore_barrier()
    pltpu.sync_copy(acc_sh.at[pl.ds(row0, RPT)], out_hbm.at[c, pl.ds(row0, RPT)])


def _make_sc_agg(d):
    @functools.partial(
        pl.kernel,
        out_type=jax.ShapeDtypeStruct((NC, N_PAD, d), jnp.float32),
        mesh=_SC_MESH,
        scratch_types=[
            pltpu.VMEM((K, CHUNK), jnp.int32),
            pltpu.VMEM((K, CHUNK), jnp.int32),
            pltpu.VMEM((CHUNK, d), jnp.float32),
            pltpu.SemaphoreType.DMA,
            pltpu.VMEM_SHARED((N_PAD, d), jnp.float32),
        ],
    )
    def agg(hp_hbm, src_hbm, dst_hbm, zeros_hbm, out_hbm,
            src_v, dst_v, rows_v, sem, acc_sh):
        _agg_body(hp_hbm, src_hbm, dst_hbm, zeros_hbm, out_hbm,
                  src_v, dst_v, rows_v, sem, acc_sh)

    return agg


_sc_agg_hid = _make_sc_agg(D_HID)
_sc_agg_out = _make_sc_agg(D_OUT)

_GRID = N_PAD // CHUNK


def _tc1_body(x_ref, w1_ref, c0_ref, c1_ref, hp_ref, dinv_ref):
    deg = 1.0 + c0_ref[...] + c1_ref[...]
    dv = lax.rsqrt(deg)
    dinv_ref[...] = dv
    hp_ref[...] = jnp.dot(x_ref[...], w1_ref[...],
                          preferred_element_type=jnp.float32) * dv[:, None]


def _tc1(x_pad, w1, c0, c1):
    return pl.pallas_call(
        _tc1_body,
        grid=(_GRID,),
        in_specs=[
            pl.BlockSpec((CHUNK, D_IN), lambda i: (i, 0)),
            pl.BlockSpec((D_IN, D_HID), lambda i: (0, 0)),
            pl.BlockSpec((CHUNK,), lambda i: (i,)),
            pl.BlockSpec((CHUNK,), lambda i: (i,)),
        ],
        out_specs=[
            pl.BlockSpec((CHUNK, D_HID), lambda i: (i, 0)),
            pl.BlockSpec((CHUNK,), lambda i: (i,)),
        ],
        out_shape=[
            jax.ShapeDtypeStruct((N_PAD, D_HID), jnp.float32),
            jax.ShapeDtypeStruct((N_PAD,), jnp.float32),
        ],
    )(x_pad, w1, c0, c1)


def _tc2_body(p0_ref, p1_ref, hp1_ref, dinv_ref, b1_ref, w2_ref, hp2_ref):
    dv = dinv_ref[...]
    h = dv[:, None] * (p0_ref[...] + p1_ref[...] + hp1_ref[...]) + b1_ref[...][None, :]
    h = jnp.maximum(h, 0.0)
    hp2_ref[...] = jnp.dot(h, w2_ref[...],
                           preferred_element_type=jnp.float32) * dv[:, None]


def _tc2(p0, p1, hp1, dinv, b1, w2):
    return pl.pallas_call(
        _tc2_body,
        grid=(_GRID,),
        in_specs=[
            pl.BlockSpec((CHUNK, D_HID), lambda i: (i, 0)),
            pl.BlockSpec((CHUNK, D_HID), lambda i: (i, 0)),
            pl.BlockSpec((CHUNK, D_HID), lambda i: (i, 0)),
            pl.BlockSpec((CHUNK,), lambda i: (i,)),
            pl.BlockSpec((D_HID,), lambda i: (0,)),
            pl.BlockSpec((D_HID, D_OUT), lambda i: (0, 0)),
        ],
        out_specs=pl.BlockSpec((CHUNK, D_OUT), lambda i: (i, 0)),
        out_shape=jax.ShapeDtypeStruct((N_PAD, D_OUT), jnp.float32),
    )(p0, p1, hp1, dinv, b1, w2)


def _tc3_body(q0_ref, q1_ref, hp2_ref, dinv_ref, b2_ref, out_ref):
    dv = dinv_ref[...]
    out_ref[...] = dv[:, None] * (q0_ref[...] + q1_ref[...] + hp2_ref[...]) \
        + b2_ref[...][None, :]


def _tc3(q0, q1, hp2, dinv, b2):
    return pl.pallas_call(
        _tc3_body,
        grid=(_GRID,),
        in_specs=[
            pl.BlockSpec((CHUNK, D_OUT), lambda i: (i, 0)),
            pl.BlockSpec((CHUNK, D_OUT), lambda i: (i, 0)),
            pl.BlockSpec((CHUNK, D_OUT), lambda i: (i, 0)),
            pl.BlockSpec((CHUNK,), lambda i: (i,)),
            pl.BlockSpec((D_OUT,), lambda i: (0,)),
        ],
        out_specs=pl.BlockSpec((CHUNK, D_OUT), lambda i: (i, 0)),
        out_shape=jax.ShapeDtypeStruct((N_PAD, D_OUT), jnp.float32),
    )(q0, q1, hp2, dinv, b2)


def kernel(x, edge_index, W1, b1, W2, b2):
    src = edge_index[0]
    dst = edge_index[1]
    pad = jnp.full((E_PAD - E,), N, jnp.int32)
    srcp = jnp.concatenate([src, pad]).reshape(NW, K, CHUNK)
    dstp = jnp.concatenate([dst, pad]).reshape(NW, K, CHUNK)
    x_pad = jnp.pad(x, ((0, N_PAD - N), (0, 0)))
    zeros1 = jnp.zeros((N_PAD,), jnp.float32)
    zeros_h = jnp.zeros((N_PAD, D_HID), jnp.float32)
    zeros_o = jnp.zeros((N_PAD, D_OUT), jnp.float32)

    cnt = _sc_counts(dstp, zeros1)
    hp1, dinv = _tc1(x_pad, W1, cnt[0], cnt[1])
    p = _sc_agg_hid(hp1, srcp, dstp, zeros_h)
    hp2 = _tc2(p[0], p[1], hp1, dinv, b1, W2)
    q = _sc_agg_out(hp2, srcp, dstp, zeros_o)
    outp = _tc3(q[0], q[1], hp2, dinv, b2)
    return outp[:N]


# speedup vs baseline: 13.1175x; 13.1175x over previous
import functools

import jax
import jax.numpy as jnp
from jax import lax
from jax.experimental import pallas as pl
from jax.experimental.pallas import tpu as pltpu
from jax.experimental.pallas import tpu_sc as plsc

N = 10000
E = 320000
D_IN = 128
D_HID = 128
D_OUT = 64

NC = 2
NS = 16
NW = NC * NS

CHUNK = 128
K = -(-E // (NW * CHUNK))
E_PAD = NW * K * CHUNK
N_PAD = 10240
RPT = N_PAD // NS

_SC_MESH = plsc.VectorSubcoreMesh(core_axis_name="c", subcore_axis_name="s")


def _counts_body(dst_hbm, zeros_hbm, out_hbm, dst_v, ones_v, zbuf_v, cnt_sh):
    c = lax.axis_index("c")
    s = lax.axis_index("s")
    wid = c * NS + s
    pltpu.sync_copy(dst_hbm.at[wid], dst_v)
    for i in range(CHUNK // 16):
        ones_v[pl.ds(i * 16, 16)] = jnp.ones((16,), jnp.float32)
    row0 = s * RPT
    pltpu.sync_copy(zeros_hbm.at[pl.ds(row0, RPT)], zbuf_v)
    pltpu.sync_copy(zbuf_v, cnt_sh.at[pl.ds(row0, RPT)])
    plsc.subcore_barrier()

    def body(j, carry):
        pltpu.sync_copy(ones_v, cnt_sh.at[dst_v.at[j]], add=True)
        return carry

    lax.fori_loop(0, K, body, 0)
    plsc.subcore_barrier()
    pltpu.sync_copy(cnt_sh.at[pl.ds(row0, RPT)], zbuf_v)
    pltpu.sync_copy(zbuf_v, out_hbm.at[pl.ds(c * N_PAD + row0, RPT)])


@functools.partial(
    pl.kernel,
    out_type=jax.ShapeDtypeStruct((NC * N_PAD,), jnp.float32),
    mesh=_SC_MESH,
    scratch_types=[
        pltpu.VMEM((K, CHUNK), jnp.int32),
        pltpu.VMEM((CHUNK,), jnp.float32),
        pltpu.VMEM((RPT,), jnp.float32),
        pltpu.VMEM_SHARED((N_PAD,), jnp.float32),
    ],
)
def _sc_counts(dst_hbm, zeros_hbm, out_hbm, dst_v, ones_v, zbuf_v, cnt_sh):
    _counts_body(dst_hbm, zeros_hbm, out_hbm, dst_v, ones_v, zbuf_v, cnt_sh)


WCHUNK = 128
NWB = RPT // WCHUNK


def _agg_body(hp_hbm, src_hbm, dst_hbm, zeros_hbm, out_hbm,
              src_v, dst_v, rows_v, sem, acc_sh):
    c = lax.axis_index("c")
    s = lax.axis_index("s")
    wid = c * NS + s
    pltpu.sync_copy(src_hbm.at[wid], src_v)
    pltpu.sync_copy(dst_hbm.at[wid], dst_v)
    row0 = s * RPT
    pltpu.sync_copy(zeros_hbm.at[pl.ds(0, WCHUNK)], rows_v)
    for i in range(NWB):
        pltpu.sync_copy(rows_v, acc_sh.at[pl.ds(row0 + i * WCHUNK, WCHUNK)])
    plsc.subcore_barrier()

    def body(j, carry):
        pltpu.async_copy(hp_hbm.at[src_v.at[j]], rows_v, sem).wait()
        pltpu.sync_copy(rows_v, acc_sh.at[dst_v.at[j]], add=True)
        return carry

    lax.fori_loop(0, K, body, 0)
    plsc.subcore_barrier()
    for i in range(NWB):
        r = row0 + i * WCHUNK
        pltpu.sync_copy(acc_sh.at[pl.ds(r, WCHUNK)], rows_v)
        pltpu.sync_copy(rows_v, out_hbm.at[c, pl.ds(r, WCHUNK)])


def _make_sc_agg(d):
    @functools.partial(
        pl.kernel,
        out_type=jax.ShapeDtypeStruct((NC, N_PAD, d), jnp.float32),
        mesh=_SC_MESH,
        compiler_params=pltpu.CompilerParams(use_tc_tiling_on_sc=False)
        if d % 128 else None,
        scratch_types=[
            pltpu.VMEM((K, CHUNK), jnp.int32),
            pltpu.VMEM((K, CHUNK), jnp.int32),
            pltpu.VMEM((CHUNK, d), jnp.float32),
            pltpu.SemaphoreType.DMA,
            pltpu.VMEM_SHARED((N_PAD, d), jnp.float32),
        ],
    )
    def agg(hp_hbm, src_hbm, dst_hbm, zeros_hbm, out_hbm,
            src_v, dst_v, rows_v, sem, acc_sh):
        _agg_body(hp_hbm, src_hbm, dst_hbm, zeros_hbm, out_hbm,
                  src_v, dst_v, rows_v, sem, acc_sh)

    return agg


_sc_agg_hid = _make_sc_agg(D_HID)
_sc_agg_out = _make_sc_agg(D_OUT)

_GRID = N_PAD // CHUNK


def _tc1_body(x_ref, w1_ref, c0_ref, c1_ref, hp_ref, dinv_ref):
    deg = 1.0 + c0_ref[...] + c1_ref[...]
    dv = lax.rsqrt(deg)
    dinv_ref[...] = dv
    hp_ref[...] = jnp.dot(x_ref[...], w1_ref[...],
                          preferred_element_type=jnp.float32) * dv[:, None]


def _tc1(x_pad, w1, c0, c1):
    return pl.pallas_call(
        _tc1_body,
        grid=(_GRID,),
        in_specs=[
            pl.BlockSpec((CHUNK, D_IN), lambda i: (i, 0)),
            pl.BlockSpec((D_IN, D_HID), lambda i: (0, 0)),
            pl.BlockSpec((CHUNK,), lambda i: (i,)),
            pl.BlockSpec((CHUNK,), lambda i: (i,)),
        ],
        out_specs=[
            pl.BlockSpec((CHUNK, D_HID), lambda i: (i, 0)),
            pl.BlockSpec((CHUNK,), lambda i: (i,)),
        ],
        out_shape=[
            jax.ShapeDtypeStruct((N_PAD, D_HID), jnp.float32),
            jax.ShapeDtypeStruct((N_PAD,), jnp.float32),
        ],
    )(x_pad, w1, c0, c1)


def _tc2_body(p0_ref, p1_ref, hp1_ref, dinv_ref, b1_ref, w2_ref, hp2_ref):
    dv = dinv_ref[...]
    h = dv[:, None] * (p0_ref[...] + p1_ref[...] + hp1_ref[...]) + b1_ref[...][None, :]
    h = jnp.maximum(h, 0.0)
    hp2_ref[...] = jnp.dot(h, w2_ref[...],
                           preferred_element_type=jnp.float32) * dv[:, None]


def _tc2(p0, p1, hp1, dinv, b1, w2):
    return pl.pallas_call(
        _tc2_body,
        grid=(_GRID,),
        in_specs=[
            pl.BlockSpec((CHUNK, D_HID), lambda i: (i, 0)),
            pl.BlockSpec((CHUNK, D_HID), lambda i: (i, 0)),
            pl.BlockSpec((CHUNK, D_HID), lambda i: (i, 0)),
            pl.BlockSpec((CHUNK,), lambda i: (i,)),
            pl.BlockSpec((D_HID,), lambda i: (0,)),
            pl.BlockSpec((D_HID, D_OUT), lambda i: (0, 0)),
        ],
        out_specs=pl.BlockSpec((CHUNK, D_OUT), lambda i: (i, 0)),
        out_shape=jax.ShapeDtypeStruct((N_PAD, D_OUT), jnp.float32),
    )(p0, p1, hp1, dinv, b1, w2)


def _tc3_body(q0_ref, q1_ref, hp2_ref, dinv_ref, b2_ref, out_ref):
    dv = dinv_ref[...]
    out_ref[...] = dv[:, None] * (q0_ref[...] + q1_ref[...] + hp2_ref[...]) \
        + b2_ref[...][None, :]


def _tc3(q0, q1, hp2, dinv, b2):
    return pl.pallas_call(
        _tc3_body,
        grid=(_GRID,),
        in_specs=[
            pl.BlockSpec((CHUNK, D_OUT), lambda i: (i, 0)),
            pl.BlockSpec((CHUNK, D_OUT), lambda i: (i, 0)),
            pl.BlockSpec((CHUNK, D_OUT), lambda i: (i, 0)),
            pl.BlockSpec((CHUNK,), lambda i: (i,)),
            pl.BlockSpec((D_OUT,), lambda i: (0,)),
        ],
        out_specs=pl.BlockSpec((CHUNK, D_OUT), lambda i: (i, 0)),
        out_shape=jax.ShapeDtypeStruct((N_PAD, D_OUT), jnp.float32),
    )(q0, q1, hp2, dinv, b2)


def kernel(x, edge_index, W1, b1, W2, b2):
    src = edge_index[0]
    dst = edge_index[1]
    pad = jnp.full((E_PAD - E,), N, jnp.int32)
    srcp = jnp.concatenate([src, pad]).reshape(NW, K, CHUNK)
    dstp = jnp.concatenate([dst, pad]).reshape(NW, K, CHUNK)
    x_pad = jnp.pad(x, ((0, N_PAD - N), (0, 0)))
    zeros1 = jnp.zeros((N_PAD,), jnp.float32)
    zeros_h = jnp.zeros((WCHUNK, D_HID), jnp.float32)
    zeros_o = jnp.zeros((WCHUNK, D_OUT), jnp.float32)

    cnt = _sc_counts(dstp, zeros1).reshape(NC, N_PAD)
    hp1, dinv = _tc1(x_pad, W1, cnt[0], cnt[1])
    p = _sc_agg_hid(hp1, srcp, dstp, zeros_h)
    hp2 = _tc2(p[0], p[1], hp1, dinv, b1, W2)
    q = _sc_agg_out(hp2, srcp, dstp, zeros_o)
    outp = _tc3(q[0], q[1], hp2, dinv, b2)
    return outp[:N]


# speedup vs baseline: 13.2766x; 1.0121x over previous
import functools

import jax
import jax.numpy as jnp
from jax import lax
from jax.experimental import pallas as pl
from jax.experimental.pallas import tpu as pltpu
from jax.experimental.pallas import tpu_sc as plsc

N = 10000
E = 320000
D_IN = 128
D_HID = 128
D_OUT = 64

NC = 2
NS = 16
LANES = 16

CHUNK = 128
NBUF = 2
K = 160
E_PAD = NS * K * CHUNK
N_PAD = 10240
RPT = N_PAD // NS
K2 = K // NC

WCHUNK = 64
NWB = RPT // WCHUNK

_SC_MESH = plsc.VectorSubcoreMesh(core_axis_name="c", subcore_axis_name="s")
_SC_PARAMS = pltpu.CompilerParams(use_tc_tiling_on_sc=False)


def _counts_body(dst_hbm, zeros_hbm, out_hbm, dst_v, ones_v, zbuf_v, cnt_sh):
    c = lax.axis_index("c")
    s = lax.axis_index("s")
    pltpu.sync_copy(dst_hbm.at[s, pl.ds(c * K2, K2)], dst_v)
    for i in range(CHUNK // LANES):
        ones_v[pl.ds(i * LANES, LANES)] = jnp.ones((LANES,), jnp.float32)
    row0 = s * RPT
    pltpu.sync_copy(zeros_hbm.at[pl.ds(row0, RPT)], zbuf_v)
    pltpu.sync_copy(zbuf_v, cnt_sh.at[pl.ds(row0, RPT)])
    plsc.subcore_barrier()

    def body(j, carry):
        pltpu.sync_copy(ones_v, cnt_sh.at[dst_v.at[j]], add=True)
        return carry

    lax.fori_loop(0, K2, body, 0)
    plsc.subcore_barrier()
    pltpu.sync_copy(cnt_sh.at[pl.ds(row0, RPT)], zbuf_v)
    pltpu.sync_copy(zbuf_v, out_hbm.at[pl.ds(c * N_PAD + row0, RPT)])


@functools.partial(
    pl.kernel,
    out_type=jax.ShapeDtypeStruct((NC * N_PAD,), jnp.float32),
    mesh=_SC_MESH,
    scratch_types=[
        pltpu.VMEM((K2, CHUNK), jnp.int32),
        pltpu.VMEM((CHUNK,), jnp.float32),
        pltpu.VMEM((RPT,), jnp.float32),
        pltpu.VMEM_SHARED((N_PAD,), jnp.float32),
    ],
)
def _sc_counts(dst_hbm, zeros_hbm, out_hbm, dst_v, ones_v, zbuf_v, cnt_sh):
    _counts_body(dst_hbm, zeros_hbm, out_hbm, dst_v, ones_v, zbuf_v, cnt_sh)


def _agg_body(hp_hbm, src_hbm, dst_hbm, zeros_hbm, out_hbm,
              src_v, dst_v, rows, sems, acc_sh):
    c = lax.axis_index("c")
    s = lax.axis_index("s")
    pltpu.sync_copy(src_hbm.at[s], src_v)
    pltpu.sync_copy(dst_hbm.at[s], dst_v)

    @pl.when(c == 1)
    def _():
        off = jnp.full((LANES,), N_PAD, jnp.int32)

        def adj(j, carry):
            for p in range(CHUNK // LANES):
                sl = pl.ds(p * LANES, LANES)
                src_v[j, sl] = src_v[j, sl] + off
            return carry

        lax.fori_loop(0, K, adj, 0)

    row0 = s * RPT
    zbuf = rows[0].at[pl.ds(0, WCHUNK)]
    pltpu.sync_copy(zeros_hbm.at[pl.ds(0, WCHUNK)], zbuf)
    for i in range(NWB):
        pltpu.sync_copy(zbuf, acc_sh.at[pl.ds(row0 + i * WCHUNK, WCHUNK)])
    plsc.subcore_barrier()

    for b in range(NBUF):
        pltpu.async_copy(hp_hbm.at[src_v.at[b]], rows[b], sems[b])

    def outer(g, carry):
        for b in range(NBUF):
            j = g * NBUF + b
            pltpu.make_async_copy(hp_hbm.at[src_v.at[j]], rows[b],
                                  sems[b]).wait()
            pltpu.sync_copy(rows[b], acc_sh.at[dst_v.at[j]], add=True)

            @pl.when(j + NBUF < K)
            def _():
                pltpu.async_copy(hp_hbm.at[src_v.at[j + NBUF]], rows[b],
                                 sems[b])
        return carry

    lax.fori_loop(0, K // NBUF, outer, 0)
    plsc.subcore_barrier()
    for i in range(NWB):
        r = row0 + i * WCHUNK
        pltpu.sync_copy(acc_sh.at[pl.ds(r, WCHUNK)], zbuf)
        pltpu.sync_copy(zbuf, out_hbm.at[c, pl.ds(r, WCHUNK)])


def _make_sc_agg(dh):
    @functools.partial(
        pl.kernel,
        out_type=jax.ShapeDtypeStruct((NC, N_PAD, dh), jnp.float32),
        mesh=_SC_MESH,
        compiler_params=_SC_PARAMS,
        scratch_types=[
            pltpu.VMEM((K, CHUNK), jnp.int32),
            pltpu.VMEM((K, CHUNK), jnp.int32),
            [pltpu.VMEM((CHUNK, dh), jnp.float32) for _ in range(NBUF)],
            [pltpu.SemaphoreType.DMA for _ in range(NBUF)],
            pltpu.VMEM_SHARED((N_PAD, dh), jnp.float32),
        ],
    )
    def agg(hp_hbm, src_hbm, dst_hbm, zeros_hbm, out_hbm,
            src_v, dst_v, rows, sems, acc_sh):
        _agg_body(hp_hbm, src_hbm, dst_hbm, zeros_hbm, out_hbm,
                  src_v, dst_v, rows, sems, acc_sh)

    return agg


_sc_agg_l1 = _make_sc_agg(D_HID // 2)
_sc_agg_l2 = _make_sc_agg(D_OUT // 2)

TCB = 128
_GRID = N_PAD // TCB
H1 = D_HID // 2
H2 = D_OUT // 2


def _tc1_body(x_ref, w1_ref, c0_ref, c1_ref, hp_ref, hpb_ref, dinv_ref):
    deg = 1.0 + c0_ref[...] + c1_ref[...]
    dv = lax.rsqrt(deg)
    dinv_ref[...] = dv
    hp = jnp.dot(x_ref[...], w1_ref[...],
                 preferred_element_type=jnp.float32) * dv[:, None]
    hp_ref[...] = hp
    hpb_ref[0] = hp[:, :H1]
    hpb_ref[1] = hp[:, H1:]


def _tc1(x_pad, w1, c0, c1):
    return pl.pallas_call(
        _tc1_body,
        grid=(_GRID,),
        in_specs=[
            pl.BlockSpec((TCB, D_IN), lambda i: (i, 0)),
            pl.BlockSpec((D_IN, D_HID), lambda i: (0, 0)),
            pl.BlockSpec((TCB,), lambda i: (i,)),
            pl.BlockSpec((TCB,), lambda i: (i,)),
        ],
        out_specs=[
            pl.BlockSpec((TCB, D_HID), lambda i: (i, 0)),
            pl.BlockSpec((NC, TCB, H1), lambda i: (0, i, 0)),
            pl.BlockSpec((TCB,), lambda i: (i,)),
        ],
        out_shape=[
            jax.ShapeDtypeStruct((N_PAD, D_HID), jnp.float32),
            jax.ShapeDtypeStruct((NC, N_PAD, H1), jnp.float32),
            jax.ShapeDtypeStruct((N_PAD,), jnp.float32),
        ],
    )(x_pad, w1, c0, c1)


def _tc2_body(p0_ref, p1_ref, hp1_ref, dinv_ref, b1_ref, w2_ref,
              hp2_ref, hp2b_ref):
    dv = dinv_ref[...]
    agg = jnp.concatenate([p0_ref[...], p1_ref[...]], axis=1)
    h = dv[:, None] * (agg + hp1_ref[...]) + b1_ref[...][None, :]
    h = jnp.maximum(h, 0.0)
    hp2 = jnp.dot(h, w2_ref[...],
                  preferred_element_type=jnp.float32) * dv[:, None]
    hp2_ref[...] = hp2
    hp2b_ref[0] = hp2[:, :H2]
    hp2b_ref[1] = hp2[:, H2:]


def _tc2(p0, p1, hp1, dinv, b1, w2):
    return pl.pallas_call(
        _tc2_body,
        grid=(_GRID,),
        in_specs=[
            pl.BlockSpec((TCB, H1), lambda i: (i, 0)),
            pl.BlockSpec((TCB, H1), lambda i: (i, 0)),
            pl.BlockSpec((TCB, D_HID), lambda i: (i, 0)),
            pl.BlockSpec((TCB,), lambda i: (i,)),
            pl.BlockSpec((D_HID,), lambda i: (0,)),
            pl.BlockSpec((D_HID, D_OUT), lambda i: (0, 0)),
        ],
        out_specs=[
            pl.BlockSpec((TCB, D_OUT), lambda i: (i, 0)),
            pl.BlockSpec((NC, TCB, H2), lambda i: (0, i, 0)),
        ],
        out_shape=[
            jax.ShapeDtypeStruct((N_PAD, D_OUT), jnp.float32),
            jax.ShapeDtypeStruct((NC, N_PAD, H2), jnp.float32),
        ],
    )(p0, p1, hp1, dinv, b1, w2)


def _tc3_body(q0_ref, q1_ref, hp2_ref, dinv_ref, b2_ref, out_ref):
    dv = dinv_ref[...]
    agg = jnp.concatenate([q0_ref[...], q1_ref[...]], axis=1)
    out_ref[...] = dv[:, None] * (agg + hp2_ref[...]) + b2_ref[...][None, :]


def _tc3(q0, q1, hp2, dinv, b2):
    return pl.pallas_call(
        _tc3_body,
        grid=(_GRID,),
        in_specs=[
            pl.BlockSpec((TCB, H2), lambda i: (i, 0)),
            pl.BlockSpec((TCB, H2), lambda i: (i, 0)),
            pl.BlockSpec((TCB, D_OUT), lambda i: (i, 0)),
            pl.BlockSpec((TCB,), lambda i: (i,)),
            pl.BlockSpec((D_OUT,), lambda i: (0,)),
        ],
        out_specs=pl.BlockSpec((TCB, D_OUT), lambda i: (i, 0)),
        out_shape=jax.ShapeDtypeStruct((N_PAD, D_OUT), jnp.float32),
    )(q0, q1, hp2, dinv, b2)


def kernel(x, edge_index, W1, b1, W2, b2):
    src = edge_index[0]
    dst = edge_index[1]
    pad = jnp.full((E_PAD - E,), N, jnp.int32)
    srcp = jnp.concatenate([src, pad]).reshape(NS, K, CHUNK)
    dstp = jnp.concatenate([dst, pad]).reshape(NS, K, CHUNK)
    x_pad = jnp.pad(x, ((0, N_PAD - N), (0, 0)))
    zeros1 = jnp.zeros((N_PAD,), jnp.float32)
    zeros_h = jnp.zeros((WCHUNK, H1), jnp.float32)
    zeros_o = jnp.zeros((WCHUNK, H2), jnp.float32)

    cnt = _sc_counts(dstp, zeros1).reshape(NC, N_PAD)
    hp1, hp1b, dinv = _tc1(x_pad, W1, cnt[0], cnt[1])
    p = _sc_agg_l1(hp1b.reshape(NC * N_PAD, H1), srcp, dstp, zeros_h)
    hp2, hp2b = _tc2(p[0], p[1], hp1, dinv, b1, W2)
    q = _sc_agg_l2(hp2b.reshape(NC * N_PAD, H2), srcp, dstp, zeros_o)
    outp = _tc3(q[0], q[1], hp2, dinv, b2)
    return outp[:N]


# speedup vs baseline: 16.4612x; 1.2399x over previous
import functools

import jax
import jax.numpy as jnp
from jax import lax
from jax.experimental import pallas as pl
from jax.experimental.pallas import tpu as pltpu
from jax.experimental.pallas import tpu_sc as plsc

N = 10000
E = 320000
D_IN = 128
D_HID = 128
D_OUT = 64

NC = 2
NS = 16
LANES = 16

CHUNK = 128
NBUF = 2
K = 160
E_PAD = NS * K * CHUNK
N_PAD = 10240
RPT = N_PAD // NS
K2 = K // NC

WCHUNK = 64
NWB = RPT // WCHUNK

_SC_MESH = plsc.VectorSubcoreMesh(core_axis_name="c", subcore_axis_name="s")
_SC_PARAMS = pltpu.CompilerParams(use_tc_tiling_on_sc=False)


def _counts_body(dst_hbm, zeros_hbm, out_hbm, dst_v, ones_v, zbuf_v, cnt_sh):
    c = lax.axis_index("c")
    s = lax.axis_index("s")
    pltpu.sync_copy(dst_hbm.at[s, pl.ds(c * K2, K2)], dst_v)
    for i in range(CHUNK // LANES):
        ones_v[pl.ds(i * LANES, LANES)] = jnp.ones((LANES,), jnp.float32)
    row0 = s * RPT
    pltpu.sync_copy(zeros_hbm.at[pl.ds(row0, RPT)], zbuf_v)
    pltpu.sync_copy(zbuf_v, cnt_sh.at[pl.ds(row0, RPT)])
    plsc.subcore_barrier()

    def body(j, carry):
        pltpu.sync_copy(ones_v, cnt_sh.at[dst_v.at[j]], add=True)
        return carry

    lax.fori_loop(0, K2, body, 0)
    plsc.subcore_barrier()
    pltpu.sync_copy(cnt_sh.at[pl.ds(row0, RPT)], zbuf_v)
    pltpu.sync_copy(zbuf_v, out_hbm.at[pl.ds(c * N_PAD + row0, RPT)])


@functools.partial(
    pl.kernel,
    out_type=jax.ShapeDtypeStruct((NC * N_PAD,), jnp.float32),
    mesh=_SC_MESH,
    scratch_types=[
        pltpu.VMEM((K2, CHUNK), jnp.int32),
        pltpu.VMEM((CHUNK,), jnp.float32),
        pltpu.VMEM((RPT,), jnp.float32),
        pltpu.VMEM_SHARED((N_PAD,), jnp.float32),
    ],
)
def _sc_counts(dst_hbm, zeros_hbm, out_hbm, dst_v, ones_v, zbuf_v, cnt_sh):
    _counts_body(dst_hbm, zeros_hbm, out_hbm, dst_v, ones_v, zbuf_v, cnt_sh)


def _agg_body(hp_hbm, src_hbm, dst_hbm, zeros_hbm, out_hbm,
              src_v, dst_v, rows, sems, acc_sh):
    c = lax.axis_index("c")
    s = lax.axis_index("s")
    pltpu.sync_copy(src_hbm.at[s], src_v)
    pltpu.sync_copy(dst_hbm.at[s], dst_v)
    tbl = hp_hbm.at[c]
    row0 = s * RPT
    zbuf = rows[0].at[pl.ds(0, WCHUNK)]
    pltpu.sync_copy(zeros_hbm.at[pl.ds(0, WCHUNK)], zbuf)
    for i in range(NWB):
        pltpu.sync_copy(zbuf, acc_sh.at[pl.ds(row0 + i * WCHUNK, WCHUNK)])
    plsc.subcore_barrier()

    for b in range(NBUF):
        pltpu.async_copy(tbl.at[src_v.at[b]], rows[b], sems[b])

    def outer(g, carry):
        for b in range(NBUF):
            j = g * NBUF + b
            pltpu.make_async_copy(tbl.at[src_v.at[j]], rows[b],
                                  sems[b]).wait()
            pltpu.sync_copy(rows[b], acc_sh.at[dst_v.at[j]], add=True)

            @pl.when(j + NBUF < K)
            def _():
                pltpu.async_copy(tbl.at[src_v.at[j + NBUF]], rows[b],
                                 sems[b])
        return carry

    lax.fori_loop(0, K // NBUF, outer, 0)
    plsc.subcore_barrier()
    for i in range(NWB):
        r = row0 + i * WCHUNK
        pltpu.sync_copy(acc_sh.at[pl.ds(r, WCHUNK)], zbuf)
        pltpu.sync_copy(zbuf, out_hbm.at[c, pl.ds(r, WCHUNK)])


def _make_sc_agg(dh):
    @functools.partial(
        pl.kernel,
        out_type=jax.ShapeDtypeStruct((NC, N_PAD, dh), jnp.float32),
        mesh=_SC_MESH,
        compiler_params=_SC_PARAMS,
        scratch_types=[
            pltpu.VMEM((K, CHUNK), jnp.int32),
            pltpu.VMEM((K, CHUNK), jnp.int32),
            [pltpu.VMEM((CHUNK, dh), jnp.float32) for _ in range(NBUF)],
            [pltpu.SemaphoreType.DMA for _ in range(NBUF)],
            pltpu.VMEM_SHARED((N_PAD, dh), jnp.float32),
        ],
    )
    def agg(hp_hbm, src_hbm, dst_hbm, zeros_hbm, out_hbm,
            src_v, dst_v, rows, sems, acc_sh):
        _agg_body(hp_hbm, src_hbm, dst_hbm, zeros_hbm, out_hbm,
                  src_v, dst_v, rows, sems, acc_sh)

    return agg


_sc_agg_l1 = _make_sc_agg(D_HID // 2)
_sc_agg_l2 = _make_sc_agg(D_OUT // 2)

TCB = 512
_GRID = N_PAD // TCB
NBLK = N_PAD // TCB
H1 = D_HID // 2
H2 = D_OUT // 2


def _tc1_body(x_ref, w1_ref, c0_ref, c1_ref, hpb_ref, dinv_ref):
    deg = 1.0 + c0_ref[...] + c1_ref[...]
    dv = lax.rsqrt(deg)
    dinv_ref[...] = dv
    hp = jnp.dot(x_ref[...], w1_ref[...],
                 preferred_element_type=jnp.float32) * dv[:, None]
    hpb_ref[0] = hp[:, :H1]
    hpb_ref[1] = hp[:, H1:]


def _tc1(x_pad, w1, cnt):
    return pl.pallas_call(
        _tc1_body,
        grid=(_GRID,),
        in_specs=[
            pl.BlockSpec((TCB, D_IN), lambda i: (i, 0)),
            pl.BlockSpec((D_IN, D_HID), lambda i: (0, 0)),
            pl.BlockSpec((TCB,), lambda i: (i,)),
            pl.BlockSpec((TCB,), lambda i: (NBLK + i,)),
        ],
        out_specs=[
            pl.BlockSpec((NC, TCB, H1), lambda i: (0, i, 0)),
            pl.BlockSpec((TCB,), lambda i: (i,)),
        ],
        out_shape=[
            jax.ShapeDtypeStruct((NC, N_PAD, H1), jnp.float32),
            jax.ShapeDtypeStruct((N_PAD,), jnp.float32),
        ],
    )(x_pad, w1, cnt, cnt)


def _tc2_body(p_ref, hpb_ref, dinv_ref, b1_ref, w2_ref, hp2b_ref):
    dv = dinv_ref[...]
    agg = jnp.concatenate([p_ref[0] + hpb_ref[0], p_ref[1] + hpb_ref[1]],
                          axis=1)
    h = dv[:, None] * agg + b1_ref[...][None, :]
    h = jnp.maximum(h, 0.0)
    hp2 = jnp.dot(h, w2_ref[...],
                  preferred_element_type=jnp.float32) * dv[:, None]
    hp2b_ref[0] = hp2[:, :H2]
    hp2b_ref[1] = hp2[:, H2:]


def _tc2(p, hpb, dinv, b1, w2):
    return pl.pallas_call(
        _tc2_body,
        grid=(_GRID,),
        in_specs=[
            pl.BlockSpec((NC, TCB, H1), lambda i: (0, i, 0)),
            pl.BlockSpec((NC, TCB, H1), lambda i: (0, i, 0)),
            pl.BlockSpec((TCB,), lambda i: (i,)),
            pl.BlockSpec((D_HID,), lambda i: (0,)),
            pl.BlockSpec((D_HID, D_OUT), lambda i: (0, 0)),
        ],
        out_specs=pl.BlockSpec((NC, TCB, H2), lambda i: (0, i, 0)),
        out_shape=jax.ShapeDtypeStruct((NC, N_PAD, H2), jnp.float32),
    )(p, hpb, dinv, b1, w2)


def _tc3_body(q_ref, hp2b_ref, dinv_ref, b2_ref, out_ref):
    dv = dinv_ref[...]
    agg = jnp.concatenate([q_ref[0] + hp2b_ref[0], q_ref[1] + hp2b_ref[1]],
                          axis=1)
    out_ref[...] = dv[:, None] * agg + b2_ref[...][None, :]


def _tc3(q, hp2b, dinv, b2):
    return pl.pallas_call(
        _tc3_body,
        grid=(_GRID,),
        in_specs=[
            pl.BlockSpec((NC, TCB, H2), lambda i: (0, i, 0)),
            pl.BlockSpec((NC, TCB, H2), lambda i: (0, i, 0)),
            pl.BlockSpec((TCB,), lambda i: (i,)),
            pl.BlockSpec((D_OUT,), lambda i: (0,)),
        ],
        out_specs=pl.BlockSpec((TCB, D_OUT), lambda i: (i, 0)),
        out_shape=jax.ShapeDtypeStruct((N_PAD, D_OUT), jnp.float32),
    )(q, hp2b, dinv, b2)


def kernel(x, edge_index, W1, b1, W2, b2):
    src = edge_index[0]
    dst = edge_index[1]
    pad = jnp.full((E_PAD - E,), N, jnp.int32)
    srcp = jnp.concatenate([src, pad]).reshape(NS, K, CHUNK)
    dstp = jnp.concatenate([dst, pad]).reshape(NS, K, CHUNK)
    x_pad = jnp.pad(x, ((0, N_PAD - N), (0, 0)))
    zeros1 = jnp.zeros((N_PAD,), jnp.float32)
    zeros_h = jnp.zeros((WCHUNK, H1), jnp.float32)
    zeros_o = jnp.zeros((WCHUNK, H2), jnp.float32)

    cnt = _sc_counts(dstp, zeros1)
    hpb, dinv = _tc1(x_pad, W1, cnt)
    p = _sc_agg_l1(hpb, srcp, dstp, zeros_h)
    hp2b = _tc2(p, hpb, dinv, b1, W2)
    q = _sc_agg_l2(hp2b, srcp, dstp, zeros_o)
    outp = _tc3(q, hp2b, dinv, b2)
    return outp[:N]


# speedup vs baseline: 17.7881x; 1.0806x over previous
import functools

import jax
import jax.numpy as jnp
from jax import lax
from jax.experimental import pallas as pl
from jax.experimental.pallas import tpu as pltpu
from jax.experimental.pallas import tpu_sc as plsc

N = 10000
E = 320000
D_IN = 128
D_HID = 128
D_OUT = 64

NC = 2
NS = 16
LANES = 16

CHUNK = 128
NBUF = 2
K = 160
E_PAD = NS * K * CHUNK
N_PAD = 10240
RPT = N_PAD // NS
K2 = K // NC

WCHUNK = 64
NWB = RPT // WCHUNK

_SC_MESH = plsc.VectorSubcoreMesh(core_axis_name="c", subcore_axis_name="s")
_SC_PARAMS = pltpu.CompilerParams(use_tc_tiling_on_sc=False,
                                  needs_layout_passes=False)


def _counts_body(dst_hbm, zeros_hbm, out_hbm, dst_v, ones_v, zbuf_v, cnt_sh):
    c = lax.axis_index("c")
    s = lax.axis_index("s")
    pltpu.sync_copy(dst_hbm.at[s, pl.ds(c * K2, K2)], dst_v)
    for i in range(CHUNK // LANES):
        ones_v[pl.ds(i * LANES, LANES)] = jnp.ones((LANES,), jnp.float32)
    row0 = s * RPT
    pltpu.sync_copy(zeros_hbm.at[pl.ds(row0, RPT)], zbuf_v)
    pltpu.sync_copy(zbuf_v, cnt_sh.at[pl.ds(row0, RPT)])
    plsc.subcore_barrier()

    def body(j, carry):
        pltpu.sync_copy(ones_v, cnt_sh.at[dst_v.at[j]], add=True)
        return carry

    lax.fori_loop(0, K2, body, 0)
    plsc.subcore_barrier()
    pltpu.sync_copy(cnt_sh.at[pl.ds(row0, RPT)], zbuf_v)
    pltpu.sync_copy(zbuf_v, out_hbm.at[pl.ds(c * N_PAD + row0, RPT)])


@functools.partial(
    pl.kernel,
    out_type=jax.ShapeDtypeStruct((NC * N_PAD,), jnp.float32),
    mesh=_SC_MESH,
    scratch_types=[
        pltpu.VMEM((K2, CHUNK), jnp.int32),
        pltpu.VMEM((CHUNK,), jnp.float32),
        pltpu.VMEM((RPT,), jnp.float32),
        pltpu.VMEM_SHARED((N_PAD,), jnp.float32),
    ],
)
def _sc_counts(dst_hbm, zeros_hbm, out_hbm, dst_v, ones_v, zbuf_v, cnt_sh):
    _counts_body(dst_hbm, zeros_hbm, out_hbm, dst_v, ones_v, zbuf_v, cnt_sh)


UNROLL = 4


def _agg_body(dh, hp_hbm, src_hbm, dst_hbm, zeros_hbm, out_hbm,
              src_v, dst_v, grows, frow, sems, acc_sh):
    dhw = dh // 2
    c = lax.axis_index("c")
    s = lax.axis_index("s")
    pltpu.sync_copy(src_hbm.at[s], src_v)
    pltpu.sync_copy(dst_hbm.at[s], dst_v)
    tbl = hp_hbm.at[c]
    row0 = s * RPT
    zbuf = frow.at[pl.ds(0, WCHUNK)]
    pltpu.sync_copy(zeros_hbm.at[pl.ds(0, WCHUNK)], zbuf)
    for i in range(NWB):
        pltpu.sync_copy(zbuf, acc_sh.at[pl.ds(row0 + i * WCHUNK, WCHUNK)])
    plsc.subcore_barrier()

    mask = jnp.int32(-65536)

    def unpack_rows(gbuf):
        def step(i, carry):
            for k in range(UNROLL):
                r = i * UNROLL + k
                for g in range(dhw // LANES):
                    v = gbuf[r, pl.ds(LANES * g, LANES)]
                    frow[r, pl.ds(2 * LANES * g, LANES)] = plsc.bitcast(
                        jnp.left_shift(v, 16), jnp.float32)
                    frow[r, pl.ds(2 * LANES * g + LANES, LANES)] = plsc.bitcast(
                        jnp.bitwise_and(v, mask), jnp.float32)
            return carry

        lax.fori_loop(0, CHUNK // UNROLL, step, 0)

    for b in range(NBUF):
        pltpu.async_copy(tbl.at[src_v.at[b]], grows[b], sems[b])

    def outer(g, carry):
        for b in range(NBUF):
            j = g * NBUF + b
            pltpu.make_async_copy(tbl.at[src_v.at[j]], grows[b],
                                  sems[b]).wait()
            unpack_rows(grows[b])

            @pl.when(j + NBUF < K)
            def _():
                pltpu.async_copy(tbl.at[src_v.at[j + NBUF]], grows[b],
                                 sems[b])

            pltpu.sync_copy(frow, acc_sh.at[dst_v.at[j]], add=True)
        return carry

    lax.fori_loop(0, K // NBUF, outer, 0)
    plsc.subcore_barrier()
    for i in range(NWB):
        r = row0 + i * WCHUNK
        pltpu.sync_copy(acc_sh.at[pl.ds(r, WCHUNK)], zbuf)
        pltpu.sync_copy(zbuf, out_hbm.at[c, pl.ds(r, WCHUNK)])


def _make_sc_agg(dh):
    @functools.partial(
        pl.kernel,
        out_type=jax.ShapeDtypeStruct((NC, N_PAD, dh), jnp.float32),
        mesh=_SC_MESH,
        compiler_params=_SC_PARAMS,
        scratch_types=[
            pltpu.VMEM((K, CHUNK), jnp.int32),
            pltpu.VMEM((K, CHUNK), jnp.int32),
            [pltpu.VMEM((CHUNK, dh // 2), jnp.int32) for _ in range(NBUF)],
            pltpu.VMEM((CHUNK, dh), jnp.float32),
            [pltpu.SemaphoreType.DMA for _ in range(NBUF)],
            pltpu.VMEM_SHARED((N_PAD, dh), jnp.float32),
        ],
    )
    def agg(hp_hbm, src_hbm, dst_hbm, zeros_hbm, out_hbm,
            src_v, dst_v, grows, frow, sems, acc_sh):
        _agg_body(dh, hp_hbm, src_hbm, dst_hbm, zeros_hbm, out_hbm,
                  src_v, dst_v, grows, frow, sems, acc_sh)

    return agg


_sc_agg_l1 = _make_sc_agg(D_HID // 2)
_sc_agg_l2 = _make_sc_agg(D_OUT // 2)

TCB = 512
_GRID = N_PAD // TCB
NBLK = N_PAD // TCB
H1 = D_HID // 2
H2 = D_OUT // 2


def _bf16_bits(u):
    r = u + 0x7FFF + jnp.bitwise_and(lax.shift_right_logical(u, 16), 1)
    return lax.shift_right_logical(r, 16)


def _pack_half(t, d):
    u = lax.bitcast_convert_type(t, jnp.int32)
    parts = []
    for g in range(d // 32):
        a = _bf16_bits(u[:, 32 * g:32 * g + 16])
        b = _bf16_bits(u[:, 32 * g + 16:32 * g + 32])
        parts.append(jnp.bitwise_or(a, lax.shift_left(b, 16)))
    return jnp.concatenate(parts, axis=1) if len(parts) > 1 else parts[0]


def _tc1_body(x_ref, w1_ref, c0_ref, c1_ref, hp_ref, hpb_ref, dinv_ref):
    deg = 1.0 + c0_ref[...] + c1_ref[...]
    dv = lax.rsqrt(deg)
    dinv_ref[...] = dv
    hp = jnp.dot(x_ref[...], w1_ref[...],
                 preferred_element_type=jnp.float32) * dv[:, None]
    hp_ref[...] = hp
    hpb_ref[0] = _pack_half(hp[:, :H1], H1)
    hpb_ref[1] = _pack_half(hp[:, H1:], H1)


def _tc1(x_pad, w1, cnt):
    return pl.pallas_call(
        _tc1_body,
        grid=(_GRID,),
        in_specs=[
            pl.BlockSpec((TCB, D_IN), lambda i: (i, 0)),
            pl.BlockSpec((D_IN, D_HID), lambda i: (0, 0)),
            pl.BlockSpec((TCB,), lambda i: (i,)),
            pl.BlockSpec((TCB,), lambda i: (NBLK + i,)),
        ],
        out_specs=[
            pl.BlockSpec((TCB, D_HID), lambda i: (i, 0)),
            pl.BlockSpec((NC, TCB, H1 // 2), lambda i: (0, i, 0)),
            pl.BlockSpec((TCB,), lambda i: (i,)),
        ],
        out_shape=[
            jax.ShapeDtypeStruct((N_PAD, D_HID), jnp.float32),
            jax.ShapeDtypeStruct((NC, N_PAD, H1 // 2), jnp.int32),
            jax.ShapeDtypeStruct((N_PAD,), jnp.float32),
        ],
    )(x_pad, w1, cnt, cnt)


def _tc2_body(p_ref, hp1_ref, dinv_ref, b1_ref, w2_ref, hp2_ref, hp2b_ref):
    dv = dinv_ref[...]
    agg = jnp.concatenate([p_ref[0], p_ref[1]], axis=1) + hp1_ref[...]
    h = dv[:, None] * agg + b1_ref[...][None, :]
    h = jnp.maximum(h, 0.0)
    hp2 = jnp.dot(h, w2_ref[...],
                  preferred_element_type=jnp.float32) * dv[:, None]
    hp2_ref[...] = hp2
    hp2b_ref[0] = _pack_half(hp2[:, :H2], H2)
    hp2b_ref[1] = _pack_half(hp2[:, H2:], H2)


def _tc2(p, hp1, dinv, b1, w2):
    return pl.pallas_call(
        _tc2_body,
        grid=(_GRID,),
        in_specs=[
            pl.BlockSpec((NC, TCB, H1), lambda i: (0, i, 0)),
            pl.BlockSpec((TCB, D_HID), lambda i: (i, 0)),
            pl.BlockSpec((TCB,), lambda i: (i,)),
            pl.BlockSpec((D_HID,), lambda i: (0,)),
            pl.BlockSpec((D_HID, D_OUT), lambda i: (0, 0)),
        ],
        out_specs=[
            pl.BlockSpec((TCB, D_OUT), lambda i: (i, 0)),
            pl.BlockSpec((NC, TCB, H2 // 2), lambda i: (0, i, 0)),
        ],
        out_shape=[
            jax.ShapeDtypeStruct((N_PAD, D_OUT), jnp.float32),
            jax.ShapeDtypeStruct((NC, N_PAD, H2 // 2), jnp.int32),
        ],
    )(p, hp1, dinv, b1, w2)


def _tc3_body(q_ref, hp2_ref, dinv_ref, b2_ref, out_ref):
    dv = dinv_ref[...]
    agg = jnp.concatenate([q_ref[0], q_ref[1]], axis=1) + hp2_ref[...]
    out_ref[...] = dv[:, None] * agg + b2_ref[...][None, :]


def _tc3(q, hp2, dinv, b2):
    return pl.pallas_call(
        _tc3_body,
        grid=(_GRID,),
        in_specs=[
            pl.BlockSpec((NC, TCB, H2), lambda i: (0, i, 0)),
            pl.BlockSpec((TCB, D_OUT), lambda i: (i, 0)),
            pl.BlockSpec((TCB,), lambda i: (i,)),
            pl.BlockSpec((D_OUT,), lambda i: (0,)),
        ],
        out_specs=pl.BlockSpec((TCB, D_OUT), lambda i: (i, 0)),
        out_shape=jax.ShapeDtypeStruct((N_PAD, D_OUT), jnp.float32),
    )(q, hp2, dinv, b2)


def kernel(x, edge_index, W1, b1, W2, b2):
    src = edge_index[0]
    dst = edge_index[1]
    pad = jnp.full((E_PAD - E,), N, jnp.int32)
    srcp = jnp.concatenate([src, pad]).reshape(NS, K, CHUNK)
    dstp = jnp.concatenate([dst, pad]).reshape(NS, K, CHUNK)
    x_pad = jnp.pad(x, ((0, N_PAD - N), (0, 0)))
    zeros1 = jnp.zeros((N_PAD,), jnp.float32)
    zeros_h = jnp.zeros((WCHUNK, H1), jnp.float32)
    zeros_o = jnp.zeros((WCHUNK, H2), jnp.float32)

    cnt = _sc_counts(dstp, zeros1)
    hp1, hpb, dinv = _tc1(x_pad, W1, cnt)
    p = _sc_agg_l1(hpb, srcp, dstp, zeros_h)
    hp2, hp2b = _tc2(p, hp1, dinv, b1, W2)
    q = _sc_agg_l2(hp2b, srcp, dstp, zeros_o)
    outp = _tc3(q, hp2, dinv, b2)
    return outp[:N]


# speedup vs baseline: 18.5947x; 1.0453x over previous
import functools

import jax
import jax.numpy as jnp
from jax import lax
from jax.experimental import pallas as pl
from jax.experimental.pallas import tpu as pltpu
from jax.experimental.pallas import tpu_sc as plsc

N = 10000
E = 320000
D_IN = 128
D_HID = 128
D_OUT = 64

NC = 2
NS = 16
NW = NC * NS
LANES = 16

CHUNK = 128
NBUF = 2
K = 80
E_PAD = NW * K * CHUNK
N_PAD = 10240
RPT = N_PAD // NS

WCHUNK = 64
NWB = RPT // WCHUNK

_SC_MESH = plsc.VectorSubcoreMesh(core_axis_name="c", subcore_axis_name="s")
_SC_PARAMS = pltpu.CompilerParams(use_tc_tiling_on_sc=False,
                                  needs_layout_passes=False)


def _counts_body(dst_hbm, zeros_hbm, out_hbm, dst_v, ones_v, zbuf_v, cnt_sh):
    c = lax.axis_index("c")
    s = lax.axis_index("s")
    wid = c * NS + s
    pltpu.sync_copy(dst_hbm.at[wid], dst_v)
    for i in range(CHUNK // LANES):
        ones_v[pl.ds(i * LANES, LANES)] = jnp.ones((LANES,), jnp.float32)
    row0 = s * RPT
    pltpu.sync_copy(zeros_hbm.at[pl.ds(row0, RPT)], zbuf_v)
    pltpu.sync_copy(zbuf_v, cnt_sh.at[pl.ds(row0, RPT)])
    plsc.subcore_barrier()

    def body(j, carry):
        pltpu.sync_copy(ones_v, cnt_sh.at[dst_v.at[j]], add=True)
        return carry

    lax.fori_loop(0, K, body, 0)
    plsc.subcore_barrier()
    pltpu.sync_copy(cnt_sh.at[pl.ds(row0, RPT)], zbuf_v)
    pltpu.sync_copy(zbuf_v, out_hbm.at[pl.ds(c * N_PAD + row0, RPT)])


@functools.partial(
    pl.kernel,
    out_type=jax.ShapeDtypeStruct((NC * N_PAD,), jnp.float32),
    mesh=_SC_MESH,
    scratch_types=[
        pltpu.VMEM((K, CHUNK), jnp.int32),
        pltpu.VMEM((CHUNK,), jnp.float32),
        pltpu.VMEM((RPT,), jnp.float32),
        pltpu.VMEM_SHARED((N_PAD,), jnp.float32),
    ],
)
def _sc_counts(dst_hbm, zeros_hbm, out_hbm, dst_v, ones_v, zbuf_v, cnt_sh):
    _counts_body(dst_hbm, zeros_hbm, out_hbm, dst_v, ones_v, zbuf_v, cnt_sh)


def _agg_body(hp_hbm, src_hbm, dst_hbm, zeros_hbm, out_hbm,
              src_v, dst_v, rows, sems, acc_sh):
    c = lax.axis_index("c")
    s = lax.axis_index("s")
    wid = c * NS + s
    pltpu.sync_copy(src_hbm.at[wid], src_v)
    pltpu.sync_copy(dst_hbm.at[wid], dst_v)
    row0 = s * RPT
    zbuf = rows[0].at[pl.ds(0, WCHUNK)]
    pltpu.sync_copy(zeros_hbm.at[pl.ds(0, WCHUNK)], zbuf)
    for i in range(NWB):
        pltpu.sync_copy(zbuf, acc_sh.at[pl.ds(row0 + i * WCHUNK, WCHUNK)])
    plsc.subcore_barrier()

    for b in range(NBUF):
        pltpu.async_copy(hp_hbm.at[src_v.at[b]], rows[b], sems[b])

    def outer(g, carry):
        for b in range(NBUF):
            j = g * NBUF + b
            pltpu.make_async_copy(hp_hbm.at[src_v.at[j]], rows[b],
                                  sems[b]).wait()
            pltpu.sync_copy(rows[b], acc_sh.at[dst_v.at[j]], add=True)

            @pl.when(j + NBUF < K)
            def _():
                pltpu.async_copy(hp_hbm.at[src_v.at[j + NBUF]], rows[b],
                                 sems[b])
        return carry

    lax.fori_loop(0, K // NBUF, outer, 0)
    plsc.subcore_barrier()
    for i in range(NWB):
        r = row0 + i * WCHUNK
        pltpu.sync_copy(acc_sh.at[pl.ds(r, WCHUNK)], zbuf)
        pltpu.sync_copy(zbuf, out_hbm.at[c, pl.ds(r, WCHUNK)])


def _make_sc_agg(d):
    @functools.partial(
        pl.kernel,
        out_type=jax.ShapeDtypeStruct((NC, N_PAD, d), jnp.bfloat16),
        mesh=_SC_MESH,
        compiler_params=_SC_PARAMS,
        scratch_types=[
            pltpu.VMEM((K, CHUNK), jnp.int32),
            pltpu.VMEM((K, CHUNK), jnp.int32),
            [pltpu.VMEM((CHUNK, d), jnp.bfloat16) for _ in range(NBUF)],
            [pltpu.SemaphoreType.DMA for _ in range(NBUF)],
            pltpu.VMEM_SHARED((N_PAD, d), jnp.bfloat16),
        ],
    )
    def agg(hp_hbm, src_hbm, dst_hbm, zeros_hbm, out_hbm,
            src_v, dst_v, rows, sems, acc_sh):
        _agg_body(hp_hbm, src_hbm, dst_hbm, zeros_hbm, out_hbm,
                  src_v, dst_v, rows, sems, acc_sh)

    return agg


_sc_agg_l1 = _make_sc_agg(D_HID)
_sc_agg_l2 = _make_sc_agg(D_OUT)

TCB = 512
_GRID = N_PAD // TCB
NBLK = N_PAD // TCB


def _tc1_body(x_ref, w1_ref, c0_ref, c1_ref, hp_ref, hpb_ref, dinv_ref):
    deg = 1.0 + c0_ref[...] + c1_ref[...]
    dv = lax.rsqrt(deg)
    dinv_ref[...] = dv
    hp = jnp.dot(x_ref[...], w1_ref[...],
                 preferred_element_type=jnp.float32) * dv[:, None]
    hp_ref[...] = hp
    hpb_ref[...] = hp.astype(jnp.bfloat16)


def _tc1(x_pad, w1, cnt):
    return pl.pallas_call(
        _tc1_body,
        grid=(_GRID,),
        in_specs=[
            pl.BlockSpec((TCB, D_IN), lambda i: (i, 0)),
            pl.BlockSpec((D_IN, D_HID), lambda i: (0, 0)),
            pl.BlockSpec((TCB,), lambda i: (i,)),
            pl.BlockSpec((TCB,), lambda i: (NBLK + i,)),
        ],
        out_specs=[
            pl.BlockSpec((TCB, D_HID), lambda i: (i, 0)),
            pl.BlockSpec((TCB, D_HID), lambda i: (i, 0)),
            pl.BlockSpec((TCB,), lambda i: (i,)),
        ],
        out_shape=[
            jax.ShapeDtypeStruct((N_PAD, D_HID), jnp.float32),
            jax.ShapeDtypeStruct((N_PAD, D_HID), jnp.bfloat16),
            jax.ShapeDtypeStruct((N_PAD,), jnp.float32),
        ],
    )(x_pad, w1, cnt, cnt)


def _tc2_body(p_ref, hp1_ref, dinv_ref, b1_ref, w2_ref, hp2_ref, hp2b_ref):
    dv = dinv_ref[...]
    agg = (p_ref[0].astype(jnp.float32) + p_ref[1].astype(jnp.float32)
           + hp1_ref[...])
    h = dv[:, None] * agg + b1_ref[...][None, :]
    h = jnp.maximum(h, 0.0)
    hp2 = jnp.dot(h, w2_ref[...],
                  preferred_element_type=jnp.float32) * dv[:, None]
    hp2_ref[...] = hp2
    hp2b_ref[...] = hp2.astype(jnp.bfloat16)


def _tc2(p, hp1, dinv, b1, w2):
    return pl.pallas_call(
        _tc2_body,
        grid=(_GRID,),
        in_specs=[
            pl.BlockSpec((NC, TCB, D_HID), lambda i: (0, i, 0)),
            pl.BlockSpec((TCB, D_HID), lambda i: (i, 0)),
            pl.BlockSpec((TCB,), lambda i: (i,)),
            pl.BlockSpec((D_HID,), lambda i: (0,)),
            pl.BlockSpec((D_HID, D_OUT), lambda i: (0, 0)),
        ],
        out_specs=[
            pl.BlockSpec((TCB, D_OUT), lambda i: (i, 0)),
            pl.BlockSpec((TCB, D_OUT), lambda i: (i, 0)),
        ],
        out_shape=[
            jax.ShapeDtypeStruct((N_PAD, D_OUT), jnp.float32),
            jax.ShapeDtypeStruct((N_PAD, D_OUT), jnp.bfloat16),
        ],
    )(p, hp1, dinv, b1, w2)


def _tc3_body(q_ref, hp2_ref, dinv_ref, b2_ref, out_ref):
    dv = dinv_ref[...]
    agg = (q_ref[0].astype(jnp.float32) + q_ref[1].astype(jnp.float32)
           + hp2_ref[...])
    out_ref[...] = dv[:, None] * agg + b2_ref[...][None, :]


def _tc3(q, hp2, dinv, b2):
    return pl.pallas_call(
        _tc3_body,
        grid=(_GRID,),
        in_specs=[
            pl.BlockSpec((NC, TCB, D_OUT), lambda i: (0, i, 0)),
            pl.BlockSpec((TCB, D_OUT), lambda i: (i, 0)),
            pl.BlockSpec((TCB,), lambda i: (i,)),
            pl.BlockSpec((D_OUT,), lambda i: (0,)),
        ],
        out_specs=pl.BlockSpec((TCB, D_OUT), lambda i: (i, 0)),
        out_shape=jax.ShapeDtypeStruct((N_PAD, D_OUT), jnp.float32),
    )(q, hp2, dinv, b2)


def kernel(x, edge_index, W1, b1, W2, b2):
    src = edge_index[0]
    dst = edge_index[1]
    pad = jnp.full((E_PAD - E,), N, jnp.int32)
    srcp = jnp.concatenate([src, pad]).reshape(NW, K, CHUNK)
    dstp = jnp.concatenate([dst, pad]).reshape(NW, K, CHUNK)
    x_pad = jnp.pad(x, ((0, N_PAD - N), (0, 0)))
    zeros1 = jnp.zeros((N_PAD,), jnp.float32)
    zeros_h = jnp.zeros((WCHUNK, D_HID), jnp.bfloat16)
    zeros_o = jnp.zeros((WCHUNK, D_OUT), jnp.bfloat16)

    cnt = _sc_counts(dstp, zeros1)
    hp1, hpb, dinv = _tc1(x_pad, W1, cnt)
    p = _sc_agg_l1(hpb, srcp, dstp, zeros_h)
    hp2, hp2b = _tc2(p, hp1, dinv, b1, W2)
    q = _sc_agg_l2(hp2b, srcp, dstp, zeros_o)
    outp = _tc3(q, hp2, dinv, b2)
    return outp[:N]


# speedup vs baseline: 20.4361x; 1.0990x over previous
import functools

import jax
import jax.numpy as jnp
from jax import lax
from jax.experimental import pallas as pl
from jax.experimental.pallas import tpu as pltpu
from jax.experimental.pallas import tpu_sc as plsc

N = 10000
E = 320000
D_IN = 128
D_HID = 128
D_OUT = 64

NC = 2
NS = 16
NW = NC * NS
LANES = 16

CHUNK = 128
NBUF = 2
K = 80
E_PAD = NW * K * CHUNK
N_PAD = 10240
RPT = N_PAD // NS

WCHUNK = 64
NWB = RPT // WCHUNK

_SC_MESH = plsc.VectorSubcoreMesh(core_axis_name="c", subcore_axis_name="s")
_SC_PARAMS = pltpu.CompilerParams(use_tc_tiling_on_sc=False,
                                  needs_layout_passes=False)


def _counts_body(dst_hbm, zeros_hbm, out_hbm, dst_v, ones_v, zbuf_v, cnt_sh):
    c = lax.axis_index("c")
    s = lax.axis_index("s")
    wid = c * NS + s
    pltpu.sync_copy(dst_hbm.at[wid], dst_v)
    for i in range(CHUNK // LANES):
        ones_v[pl.ds(i * LANES, LANES)] = jnp.ones((LANES,), jnp.float32)
    row0 = s * RPT
    pltpu.sync_copy(zeros_hbm.at[pl.ds(row0, RPT)], zbuf_v)
    pltpu.sync_copy(zbuf_v, cnt_sh.at[pl.ds(row0, RPT)])
    plsc.subcore_barrier()

    def body(j, carry):
        pltpu.sync_copy(ones_v, cnt_sh.at[dst_v.at[j]], add=True)
        return carry

    lax.fori_loop(0, K, body, 0)
    plsc.subcore_barrier()
    pltpu.sync_copy(cnt_sh.at[pl.ds(row0, RPT)], zbuf_v)
    pltpu.sync_copy(zbuf_v, out_hbm.at[pl.ds(c * N_PAD + row0, RPT)])


@functools.partial(
    pl.kernel,
    out_type=jax.ShapeDtypeStruct((NC * N_PAD,), jnp.float32),
    mesh=_SC_MESH,
    scratch_types=[
        pltpu.VMEM((K, CHUNK), jnp.int32),
        pltpu.VMEM((CHUNK,), jnp.float32),
        pltpu.VMEM((RPT,), jnp.float32),
        pltpu.VMEM_SHARED((N_PAD,), jnp.float32),
    ],
)
def _sc_counts(dst_hbm, zeros_hbm, out_hbm, dst_v, ones_v, zbuf_v, cnt_sh):
    _counts_body(dst_hbm, zeros_hbm, out_hbm, dst_v, ones_v, zbuf_v, cnt_sh)


def _agg_body(hp_hbm, src_hbm, dst_hbm, zeros_hbm, out_hbm,
              src_v, dst_v, rows, sems, acc_sh):
    c = lax.axis_index("c")
    s = lax.axis_index("s")
    wid = c * NS + s
    pltpu.sync_copy(src_hbm.at[wid], src_v)
    pltpu.sync_copy(dst_hbm.at[wid], dst_v)
    tbl = hp_hbm.at[c]
    row0 = s * RPT
    zbuf = rows[0].at[pl.ds(0, WCHUNK)]
    pltpu.sync_copy(zeros_hbm.at[pl.ds(0, WCHUNK)], zbuf)
    for i in range(NWB):
        pltpu.sync_copy(zbuf, acc_sh.at[pl.ds(row0 + i * WCHUNK, WCHUNK)])
    plsc.subcore_barrier()

    for b in range(NBUF):
        pltpu.async_copy(tbl.at[src_v.at[b]], rows[b], sems[b])

    def outer(g, carry):
        for b in range(NBUF):
            j = g * NBUF + b
            pltpu.make_async_copy(tbl.at[src_v.at[j]], rows[b],
                                  sems[b]).wait()
            pltpu.sync_copy(rows[b], acc_sh.at[dst_v.at[j]], add=True)

            @pl.when(j + NBUF < K)
            def _():
                pltpu.async_copy(tbl.at[src_v.at[j + NBUF]], rows[b],
                                 sems[b])
        return carry

    lax.fori_loop(0, K // NBUF, outer, 0)
    plsc.subcore_barrier()
    for i in range(NWB):
        r = row0 + i * WCHUNK
        pltpu.sync_copy(acc_sh.at[pl.ds(r, WCHUNK)], zbuf)
        pltpu.sync_copy(zbuf, out_hbm.at[c, pl.ds(r, WCHUNK)])


def _make_sc_agg(d):
    @functools.partial(
        pl.kernel,
        out_type=jax.ShapeDtypeStruct((NC, N_PAD, d), jnp.bfloat16),
        mesh=_SC_MESH,
        compiler_params=_SC_PARAMS,
        scratch_types=[
            pltpu.VMEM((K, CHUNK), jnp.int32),
            pltpu.VMEM((K, CHUNK), jnp.int32),
            [pltpu.VMEM((CHUNK, d), jnp.bfloat16) for _ in range(NBUF)],
            [pltpu.SemaphoreType.DMA for _ in range(NBUF)],
            pltpu.VMEM_SHARED((N_PAD, d), jnp.bfloat16),
        ],
    )
    def agg(hp_hbm, src_hbm, dst_hbm, zeros_hbm, out_hbm,
            src_v, dst_v, rows, sems, acc_sh):
        _agg_body(hp_hbm, src_hbm, dst_hbm, zeros_hbm, out_hbm,
                  src_v, dst_v, rows, sems, acc_sh)

    return agg


_sc_agg_l1 = _make_sc_agg(D_HID)
_sc_agg_l2 = _make_sc_agg(D_OUT)

TCB = 512
_GRID = N_PAD // TCB
NBLK = N_PAD // TCB


def _tc1_body(x_ref, w1_ref, c0_ref, c1_ref, hp_ref, hpb_ref, dinv_ref):
    deg = 1.0 + c0_ref[...] + c1_ref[...]
    dv = lax.rsqrt(deg)
    dinv_ref[...] = dv
    hp = jnp.dot(x_ref[...], w1_ref[...],
                 preferred_element_type=jnp.float32) * dv[:, None]
    hp_ref[...] = hp
    bf = hp.astype(jnp.bfloat16)
    hpb_ref[0] = bf
    hpb_ref[1] = bf


def _tc1(x_pad, w1, cnt):
    return pl.pallas_call(
        _tc1_body,
        grid=(_GRID,),
        in_specs=[
            pl.BlockSpec((TCB, D_IN), lambda i: (i, 0)),
            pl.BlockSpec((D_IN, D_HID), lambda i: (0, 0)),
            pl.BlockSpec((TCB,), lambda i: (i,)),
            pl.BlockSpec((TCB,), lambda i: (NBLK + i,)),
        ],
        out_specs=[
            pl.BlockSpec((TCB, D_HID), lambda i: (i, 0)),
            pl.BlockSpec((NC, TCB, D_HID), lambda i: (0, i, 0)),
            pl.BlockSpec((TCB,), lambda i: (i,)),
        ],
        out_shape=[
            jax.ShapeDtypeStruct((N_PAD, D_HID), jnp.float32),
            jax.ShapeDtypeStruct((NC, N_PAD, D_HID), jnp.bfloat16),
            jax.ShapeDtypeStruct((N_PAD,), jnp.float32),
        ],
    )(x_pad, w1, cnt, cnt)


def _tc2_body(p_ref, hp1_ref, dinv_ref, b1_ref, w2_ref, hp2_ref, hp2b_ref):
    dv = dinv_ref[...]
    agg = (p_ref[0].astype(jnp.float32) + p_ref[1].astype(jnp.float32)
           + hp1_ref[...])
    h = dv[:, None] * agg + b1_ref[...][None, :]
    h = jnp.maximum(h, 0.0)
    hp2 = jnp.dot(h, w2_ref[...],
                  preferred_element_type=jnp.float32) * dv[:, None]
    hp2_ref[...] = hp2
    bf2 = hp2.astype(jnp.bfloat16)
    hp2b_ref[0] = bf2
    hp2b_ref[1] = bf2


def _tc2(p, hp1, dinv, b1, w2):
    return pl.pallas_call(
        _tc2_body,
        grid=(_GRID,),
        in_specs=[
            pl.BlockSpec((NC, TCB, D_HID), lambda i: (0, i, 0)),
            pl.BlockSpec((TCB, D_HID), lambda i: (i, 0)),
            pl.BlockSpec((TCB,), lambda i: (i,)),
            pl.BlockSpec((D_HID,), lambda i: (0,)),
            pl.BlockSpec((D_HID, D_OUT), lambda i: (0, 0)),
        ],
        out_specs=[
            pl.BlockSpec((TCB, D_OUT), lambda i: (i, 0)),
            pl.BlockSpec((NC, TCB, D_OUT), lambda i: (0, i, 0)),
        ],
        out_shape=[
            jax.ShapeDtypeStruct((N_PAD, D_OUT), jnp.float32),
            jax.ShapeDtypeStruct((NC, N_PAD, D_OUT), jnp.bfloat16),
        ],
    )(p, hp1, dinv, b1, w2)


def _tc3_body(q_ref, hp2_ref, dinv_ref, b2_ref, out_ref):
    dv = dinv_ref[...]
    agg = (q_ref[0].astype(jnp.float32) + q_ref[1].astype(jnp.float32)
           + hp2_ref[...])
    out_ref[...] = dv[:, None] * agg + b2_ref[...][None, :]


def _tc3(q, hp2, dinv, b2):
    return pl.pallas_call(
        _tc3_body,
        grid=(_GRID,),
        in_specs=[
            pl.BlockSpec((NC, TCB, D_OUT), lambda i: (0, i, 0)),
            pl.BlockSpec((TCB, D_OUT), lambda i: (i, 0)),
            pl.BlockSpec((TCB,), lambda i: (i,)),
            pl.BlockSpec((D_OUT,), lambda i: (0,)),
        ],
        out_specs=pl.BlockSpec((TCB, D_OUT), lambda i: (i, 0)),
        out_shape=jax.ShapeDtypeStruct((N_PAD, D_OUT), jnp.float32),
    )(q, hp2, dinv, b2)


def kernel(x, edge_index, W1, b1, W2, b2):
    src = edge_index[0]
    dst = edge_index[1]
    pad = jnp.full((E_PAD - E,), N, jnp.int32)
    srcp = jnp.concatenate([src, pad]).reshape(NW, K, CHUNK)
    dstp = jnp.concatenate([dst, pad]).reshape(NW, K, CHUNK)
    x_pad = jnp.pad(x, ((0, N_PAD - N), (0, 0)))
    zeros1 = jnp.zeros((N_PAD,), jnp.float32)
    zeros_h = jnp.zeros((WCHUNK, D_HID), jnp.bfloat16)
    zeros_o = jnp.zeros((WCHUNK, D_OUT), jnp.bfloat16)

    cnt = _sc_counts(dstp, zeros1)
    hp1, hpb, dinv = _tc1(x_pad, W1, cnt)
    p = _sc_agg_l1(hpb, srcp, dstp, zeros_h)
    hp2, hp2b = _tc2(p, hp1, dinv, b1, W2)
    q = _sc_agg_l2(hp2b, srcp, dstp, zeros_o)
    outp = _tc3(q, hp2, dinv, b2)
    return outp[:N]


# speedup vs baseline: 20.7294x; 1.0143x over previous
import functools

import jax
import jax.numpy as jnp
from jax import lax
from jax.experimental import pallas as pl
from jax.experimental.pallas import tpu as pltpu
from jax.experimental.pallas import tpu_sc as plsc

N = 10000
E = 320000
D_IN = 128
D_HID = 128
D_OUT = 64

NC = 2
NS = 16
NW = NC * NS
LANES = 16

CHUNK = 128
NBUF = 4
K = 80
E_PAD = NW * K * CHUNK
N_PAD = 10240
RPT = N_PAD // NS

WCHUNK = 64
NWB = RPT // WCHUNK

_SC_MESH = plsc.VectorSubcoreMesh(core_axis_name="c", subcore_axis_name="s")
_SC_PARAMS = pltpu.CompilerParams(use_tc_tiling_on_sc=False,
                                  needs_layout_passes=False)


def _counts_body(dst_hbm, zeros_hbm, out_hbm, dst_v, ones_v, zbuf_v, cnt_sh):
    c = lax.axis_index("c")
    s = lax.axis_index("s")
    wid = c * NS + s
    pltpu.sync_copy(dst_hbm.at[wid], dst_v)
    for i in range(CHUNK // LANES):
        ones_v[pl.ds(i * LANES, LANES)] = jnp.ones((LANES,), jnp.float32)
    row0 = s * RPT
    pltpu.sync_copy(zeros_hbm.at[pl.ds(row0, RPT)], zbuf_v)
    pltpu.sync_copy(zbuf_v, cnt_sh.at[pl.ds(row0, RPT)])
    plsc.subcore_barrier()

    def body(j, carry):
        pltpu.sync_copy(ones_v, cnt_sh.at[dst_v.at[j]], add=True)
        return carry

    lax.fori_loop(0, K, body, 0)
    plsc.subcore_barrier()
    pltpu.sync_copy(cnt_sh.at[pl.ds(row0, RPT)], zbuf_v)
    pltpu.sync_copy(zbuf_v, out_hbm.at[pl.ds(c * N_PAD + row0, RPT)])


@functools.partial(
    pl.kernel,
    out_type=jax.ShapeDtypeStruct((NC * N_PAD,), jnp.float32),
    mesh=_SC_MESH,
    scratch_types=[
        pltpu.VMEM((K, CHUNK), jnp.int32),
        pltpu.VMEM((CHUNK,), jnp.float32),
        pltpu.VMEM((RPT,), jnp.float32),
        pltpu.VMEM_SHARED((N_PAD,), jnp.float32),
    ],
)
def _sc_counts(dst_hbm, zeros_hbm, out_hbm, dst_v, ones_v, zbuf_v, cnt_sh):
    _counts_body(dst_hbm, zeros_hbm, out_hbm, dst_v, ones_v, zbuf_v, cnt_sh)


def _agg_body(hp_hbm, src_hbm, dst_hbm, zeros_hbm, out_hbm,
              src_v, dst_v, rows, sems, acc_sh):
    c = lax.axis_index("c")
    s = lax.axis_index("s")
    wid = c * NS + s
    pltpu.sync_copy(src_hbm.at[wid], src_v)
    pltpu.sync_copy(dst_hbm.at[wid], dst_v)
    tbl = hp_hbm.at[c]
    row0 = s * RPT
    zbuf = rows[0].at[pl.ds(0, WCHUNK)]
    pltpu.sync_copy(zeros_hbm.at[pl.ds(0, WCHUNK)], zbuf)
    for i in range(NWB):
        pltpu.sync_copy(zbuf, acc_sh.at[pl.ds(row0 + i * WCHUNK, WCHUNK)])
    plsc.subcore_barrier()

    for b in range(NBUF):
        pltpu.async_copy(tbl.at[src_v.at[b]], rows[b], sems[b])

    def outer(g, carry):
        for b in range(NBUF):
            j = g * NBUF + b
            pltpu.make_async_copy(tbl.at[src_v.at[j]], rows[b],
                                  sems[b]).wait()
            pltpu.sync_copy(rows[b], acc_sh.at[dst_v.at[j]], add=True)

            @pl.when(j + NBUF < K)
            def _():
                pltpu.async_copy(tbl.at[src_v.at[j + NBUF]], rows[b],
                                 sems[b])
        return carry

    lax.fori_loop(0, K // NBUF, outer, 0)
    plsc.subcore_barrier()
    for i in range(NWB):
        r = row0 + i * WCHUNK
        pltpu.sync_copy(acc_sh.at[pl.ds(r, WCHUNK)], zbuf)
        pltpu.sync_copy(zbuf, out_hbm.at[c, pl.ds(r, WCHUNK)])


def _make_sc_agg(d):
    @functools.partial(
        pl.kernel,
        out_type=jax.ShapeDtypeStruct((NC, N_PAD, d), jnp.bfloat16),
        mesh=_SC_MESH,
        compiler_params=_SC_PARAMS,
        scratch_types=[
            pltpu.VMEM((K, CHUNK), jnp.int32),
            pltpu.VMEM((K, CHUNK), jnp.int32),
            [pltpu.VMEM((CHUNK, d), jnp.bfloat16) for _ in range(NBUF)],
            [pltpu.SemaphoreType.DMA for _ in range(NBUF)],
            pltpu.VMEM_SHARED((N_PAD, d), jnp.bfloat16),
        ],
    )
    def agg(hp_hbm, src_hbm, dst_hbm, zeros_hbm, out_hbm,
            src_v, dst_v, rows, sems, acc_sh):
        _agg_body(hp_hbm, src_hbm, dst_hbm, zeros_hbm, out_hbm,
                  src_v, dst_v, rows, sems, acc_sh)

    return agg


_sc_agg_l1 = _make_sc_agg(D_HID)
_sc_agg_l2 = _make_sc_agg(D_OUT)

TCB = 512
_GRID = N_PAD // TCB
NBLK = N_PAD // TCB


def _tc1_body(x_ref, w1_ref, c0_ref, c1_ref, hp_ref, hpb_ref, dinv_ref):
    deg = 1.0 + c0_ref[...] + c1_ref[...]
    dv = lax.rsqrt(deg)
    dinv_ref[...] = dv
    hp = jnp.dot(x_ref[...], w1_ref[...],
                 preferred_element_type=jnp.float32) * dv[:, None]
    hp_ref[...] = hp
    bf = hp.astype(jnp.bfloat16)
    hpb_ref[0] = bf
    hpb_ref[1] = bf


def _tc1(x_pad, w1, cnt):
    return pl.pallas_call(
        _tc1_body,
        grid=(_GRID,),
        in_specs=[
            pl.BlockSpec((TCB, D_IN), lambda i: (i, 0)),
            pl.BlockSpec((D_IN, D_HID), lambda i: (0, 0)),
            pl.BlockSpec((TCB,), lambda i: (i,)),
            pl.BlockSpec((TCB,), lambda i: (NBLK + i,)),
        ],
        out_specs=[
            pl.BlockSpec((TCB, D_HID), lambda i: (i, 0)),
            pl.BlockSpec((NC, TCB, D_HID), lambda i: (0, i, 0)),
            pl.BlockSpec((TCB,), lambda i: (i,)),
        ],
        out_shape=[
            jax.ShapeDtypeStruct((N_PAD, D_HID), jnp.float32),
            jax.ShapeDtypeStruct((NC, N_PAD, D_HID), jnp.bfloat16),
            jax.ShapeDtypeStruct((N_PAD,), jnp.float32),
        ],
    )(x_pad, w1, cnt, cnt)


def _tc2_body(p_ref, hp1_ref, dinv_ref, b1_ref, w2_ref, hp2_ref, hp2b_ref):
    dv = dinv_ref[...]
    agg = (p_ref[0].astype(jnp.float32) + p_ref[1].astype(jnp.float32)
           + hp1_ref[...])
    h = dv[:, None] * agg + b1_ref[...][None, :]
    h = jnp.maximum(h, 0.0)
    hp2 = jnp.dot(h, w2_ref[...],
                  preferred_element_type=jnp.float32) * dv[:, None]
    hp2_ref[...] = hp2
    bf2 = hp2.astype(jnp.bfloat16)
    hp2b_ref[0] = bf2
    hp2b_ref[1] = bf2


def _tc2(p, hp1, dinv, b1, w2):
    return pl.pallas_call(
        _tc2_body,
        grid=(_GRID,),
        in_specs=[
            pl.BlockSpec((NC, TCB, D_HID), lambda i: (0, i, 0)),
            pl.BlockSpec((TCB, D_HID), lambda i: (i, 0)),
            pl.BlockSpec((TCB,), lambda i: (i,)),
            pl.BlockSpec((D_HID,), lambda i: (0,)),
            pl.BlockSpec((D_HID, D_OUT), lambda i: (0, 0)),
        ],
        out_specs=[
            pl.BlockSpec((TCB, D_OUT), lambda i: (i, 0)),
            pl.BlockSpec((NC, TCB, D_OUT), lambda i: (0, i, 0)),
        ],
        out_shape=[
            jax.ShapeDtypeStruct((N_PAD, D_OUT), jnp.float32),
            jax.ShapeDtypeStruct((NC, N_PAD, D_OUT), jnp.bfloat16),
        ],
    )(p, hp1, dinv, b1, w2)


def _tc3_body(q_ref, hp2_ref, dinv_ref, b2_ref, out_ref):
    dv = dinv_ref[...]
    agg = (q_ref[0].astype(jnp.float32) + q_ref[1].astype(jnp.float32)
           + hp2_ref[...])
    out_ref[...] = dv[:, None] * agg + b2_ref[...][None, :]


def _tc3(q, hp2, dinv, b2):
    return pl.pallas_call(
        _tc3_body,
        grid=(_GRID,),
        in_specs=[
            pl.BlockSpec((NC, TCB, D_OUT), lambda i: (0, i, 0)),
            pl.BlockSpec((TCB, D_OUT), lambda i: (i, 0)),
            pl.BlockSpec((TCB,), lambda i: (i,)),
            pl.BlockSpec((D_OUT,), lambda i: (0,)),
        ],
        out_specs=pl.BlockSpec((TCB, D_OUT), lambda i: (i, 0)),
        out_shape=jax.ShapeDtypeStruct((N_PAD, D_OUT), jnp.float32),
    )(q, hp2, dinv, b2)


def kernel(x, edge_index, W1, b1, W2, b2):
    src = edge_index[0]
    dst = edge_index[1]
    pad = jnp.full((E_PAD - E,), N, jnp.int32)
    srcp = jnp.concatenate([src, pad]).reshape(NW, K, CHUNK)
    dstp = jnp.concatenate([dst, pad]).reshape(NW, K, CHUNK)
    x_pad = jnp.pad(x, ((0, N_PAD - N), (0, 0)))
    zeros1 = jnp.zeros((N_PAD,), jnp.float32)
    zeros_h = jnp.zeros((WCHUNK, D_HID), jnp.bfloat16)
    zeros_o = jnp.zeros((WCHUNK, D_OUT), jnp.bfloat16)

    cnt = _sc_counts(dstp, zeros1)
    hp1, hpb, dinv = _tc1(x_pad, W1, cnt)
    p = _sc_agg_l1(hpb, srcp, dstp, zeros_h)
    hp2, hp2b = _tc2(p, hp1, dinv, b1, W2)
    q = _sc_agg_l2(hp2b, srcp, dstp, zeros_o)
    outp = _tc3(q, hp2, dinv, b2)
    return outp[:N]
